# Initial kernel scaffold; baseline (speedup 1.0000x reference)
import jax
import jax.numpy as jnp
from jax.experimental import pallas as pl


def kernel(x, adj, W1, b1, W2, b2):
    raise NotImplementedError("write your pallas kernel here")


# speedup vs baseline: 36.4381x; 36.4381x over previous
import functools

import jax
import jax.numpy as jnp
from jax import lax
from jax.experimental import pallas as pl
from jax.experimental.pallas import tpu as pltpu
from jax.experimental.pallas import tpu_sc as plsc

N = 10000
F_IN = 128
HID = 16
NCLASS = 40
E = 320000

CHUNK = 128
E_ROWS = 2560
E_PAD = E_ROWS * CHUNK
N_PAD = 10240
NUM_TILES = 32
ROWS_PER_TILE = E_ROWS // NUM_TILES
NROWS_PER_TILE = N_PAD // 16


def _sc_mesh():
    return plsc.VectorSubcoreMesh(core_axis_name="c", subcore_axis_name="s")


def _sc_degree(dst2d):
    @functools.partial(
        pl.kernel,
        mesh=_sc_mesh(),
        compiler_params=pltpu.CompilerParams(use_tc_tiling_on_sc=False),
        out_type=jax.ShapeDtypeStruct((2, N_PAD), jnp.float32),
        scratch_types=[
            pltpu.VMEM((ROWS_PER_TILE, CHUNK), jnp.int32),
            pltpu.VMEM((CHUNK,), jnp.float32),
            pltpu.VMEM((NROWS_PER_TILE,), jnp.float32),
            pltpu.VMEM_SHARED((N_PAD,), jnp.float32),
        ],
    )
    def k(dst_hbm, out_hbm, idx_v, ones_v, zeros_v, acc):
        c = lax.axis_index("c")
        s = lax.axis_index("s")
        wid = s * 2 + c
        for i in range(CHUNK // 16):
            ones_v[pl.ds(i * 16, 16)] = jnp.ones((16,), jnp.float32)

        def zbody(i, _):
            zeros_v[pl.ds(i * 16, 16)] = jnp.zeros((16,), jnp.float32)
            return 0

        lax.fori_loop(0, NROWS_PER_TILE // 16, zbody, 0)
        pltpu.sync_copy(zeros_v, acc.at[pl.ds(s * NROWS_PER_TILE, NROWS_PER_TILE)])
        pltpu.sync_copy(dst_hbm.at[pl.ds(wid * ROWS_PER_TILE, ROWS_PER_TILE)], idx_v)
        plsc.subcore_barrier()

        def body(i, _):
            pltpu.sync_copy(ones_v, acc.at[idx_v.at[i]], add=True)
            return 0

        lax.fori_loop(0, ROWS_PER_TILE, body, 0)
        plsc.subcore_barrier()
        pltpu.sync_copy(
            acc.at[pl.ds(s * NROWS_PER_TILE, NROWS_PER_TILE)],
            out_hbm.at[c, pl.ds(s * NROWS_PER_TILE, NROWS_PER_TILE)],
        )

    return k(dst2d)


def _sc_scatter16(table, src2d, dst2d):
    @functools.partial(
        pl.kernel,
        mesh=_sc_mesh(),
        compiler_params=pltpu.CompilerParams(use_tc_tiling_on_sc=False),
        out_type=jax.ShapeDtypeStruct((2, N_PAD, HID), jnp.float32),
        scratch_types=[
            pltpu.VMEM((ROWS_PER_TILE, CHUNK), jnp.int32),
            pltpu.VMEM((ROWS_PER_TILE, CHUNK), jnp.int32),
            pltpu.VMEM((CHUNK, HID), jnp.float32),
            pltpu.VMEM((CHUNK, HID), jnp.float32),
            pltpu.VMEM((NROWS_PER_TILE, HID), jnp.float32),
            pltpu.VMEM_SHARED((N_PAD, HID), jnp.float32),
            pltpu.SemaphoreType.DMA,
            pltpu.SemaphoreType.DMA,
        ],
    )
    def k(tab_hbm, src_hbm, dst_hbm, out_hbm,
          src_v, dst_v, rows0_v, rows1_v, zeros_v, acc, sem0, sem1):
        c = lax.axis_index("c")
        s = lax.axis_index("s")
        wid = s * 2 + c

        def zbody(i, _):
            zeros_v[i] = jnp.zeros((HID,), jnp.float32)
            return 0

        lax.fori_loop(0, NROWS_PER_TILE, zbody, 0)
        pltpu.sync_copy(
            zeros_v, acc.at[pl.ds(s * NROWS_PER_TILE, NROWS_PER_TILE)]
        )
        pltpu.sync_copy(src_hbm.at[pl.ds(wid * ROWS_PER_TILE, ROWS_PER_TILE)], src_v)
        pltpu.sync_copy(dst_hbm.at[pl.ds(wid * ROWS_PER_TILE, ROWS_PER_TILE)], dst_v)
        plsc.subcore_barrier()

        bufs = (rows0_v, rows1_v)
        sems = (sem0, sem1)
        pltpu.async_copy(tab_hbm.at[src_v.at[0]], bufs[0], sems[0])
        pltpu.async_copy(tab_hbm.at[src_v.at[1]], bufs[1], sems[1])

        def outer(io, _):
            for b in range(2):
                i = io * 2 + b
                pltpu.make_async_copy(
                    tab_hbm.at[pl.ds(0, CHUNK)], bufs[b], sems[b]
                ).wait()
                pltpu.sync_copy(bufs[b], acc.at[dst_v.at[i]], add=True)

                @pl.when(i + 2 < ROWS_PER_TILE)
                def _():
                    pltpu.async_copy(
                        tab_hbm.at[src_v.at[i + 2]], bufs[b], sems[b]
                    )
            return 0

        lax.fori_loop(0, ROWS_PER_TILE // 2, outer, 0)
        plsc.subcore_barrier()
        pltpu.sync_copy(
            acc.at[pl.ds(s * NROWS_PER_TILE, NROWS_PER_TILE)],
            out_hbm.at[c, pl.ds(s * NROWS_PER_TILE, NROWS_PER_TILE)],
        )

    return k(table, src2d, dst2d)


def _tc_pre(x_pad, W1, deg_a, deg_b):
    def body(x_ref, w_ref, da_ref, db_ref, hp_ref, dinv_ref):
        dinv = lax.rsqrt(da_ref[...] + db_ref[...] + 1.0)
        h = jnp.dot(x_ref[...], w_ref[...], preferred_element_type=jnp.float32)
        hp_ref[...] = h * dinv
        dinv_ref[...] = dinv

    return pl.pallas_call(
        body,
        out_shape=[
            jax.ShapeDtypeStruct((N_PAD, HID), jnp.float32),
            jax.ShapeDtypeStruct((N_PAD, 1), jnp.float32),
        ],
    )(x_pad, W1, deg_a, deg_b)


def _tc_mid(s1a, s1b, hp1, dinv, b1):
    def body(sa_ref, sb_ref, hp_ref, di_ref, b_ref, g_ref):
        z = di_ref[...] * (sa_ref[...] + sb_ref[...] + hp_ref[...]) + b_ref[...]
        g_ref[...] = di_ref[...] * jnp.maximum(z, 0.0)

    return pl.pallas_call(
        body,
        out_shape=jax.ShapeDtypeStruct((N_PAD, HID), jnp.float32),
    )(s1a, s1b, hp1, dinv, b1)


def _tc_out(s2a, s2b, g, dinv, W2, b2):
    def body(sa_ref, sb_ref, g_ref, di_ref, w_ref, b_ref, out_ref):
        t = di_ref[...] * (sa_ref[...] + sb_ref[...] + g_ref[...])
        o = jnp.dot(t, w_ref[...], preferred_element_type=jnp.float32) + b_ref[...]
        m = jnp.max(o, axis=1, keepdims=True)
        lse = m + jnp.log(jnp.sum(jnp.exp(o - m), axis=1, keepdims=True))
        out_ref[...] = o - lse

    return pl.pallas_call(
        body,
        out_shape=jax.ShapeDtypeStruct((N_PAD, NCLASS), jnp.float32),
    )(s2a, s2b, g, dinv, W2, b2)


def kernel(x, adj, W1, b1, W2, b2):
    src, dst = adj[0], adj[1]
    src2d = jnp.concatenate(
        [src, jnp.zeros((E_PAD - E,), jnp.int32)]
    ).reshape(E_ROWS, CHUNK)
    dst2d = jnp.concatenate(
        [dst, jnp.full((E_PAD - E,), N, jnp.int32)]
    ).reshape(E_ROWS, CHUNK)
    x_pad = jnp.concatenate([x, jnp.zeros((N_PAD - N, F_IN), x.dtype)])

    deg2 = _sc_degree(dst2d)
    hp1, dinv = _tc_pre(x_pad, W1, deg2[0][:, None], deg2[1][:, None])
    s1 = _sc_scatter16(hp1, src2d, dst2d)
    g = _tc_mid(s1[0], s1[1], hp1, dinv, b1[None, :])
    s2 = _sc_scatter16(g, src2d, dst2d)
    out = _tc_out(s2[0], s2[1], g, dinv, W2, b2[None, :])
    return out[:N]
